# baseline (speedup 1.0000x reference)
import functools

import jax
import jax.numpy as jnp
from jax import lax
from jax.experimental import pallas as pl
from jax.experimental.pallas import tpu as pltpu
from jax.experimental.pallas import tpu_sc as plsc

_SC_CORES = 2
_SC_SUBCORES = 16
_NW = _SC_CORES * _SC_SUBCORES
_IDXW = 128


def _encode_body(nchunk, chunk, tb,
                 z_ref, vin_ref, gin_ref, bin_ref, cb_ref,
                 ze_ref, idx_ref,
                 zd_s, sumz_s, bd_s, bi_s):
    k = pl.program_id(1)

    @pl.when(k == 0)
    def _():
        v = vin_ref[...]
        norm = jnp.sqrt(jnp.sum(v * v, axis=1, keepdims=True))
        w = gin_ref[...] * v / norm
        ze = lax.dot_general(w, z_ref[0], (((1,), (0,)), ((), ())))
        ze = ze + bin_ref[...]
        ze_ref[0] = ze
        zd_s[...] = (2.0 * ze).astype(jnp.bfloat16)
        sumz_s[...] = jnp.sum(ze * ze, axis=0, keepdims=True)
        bd_s[...] = jnp.full(bd_s.shape, jnp.inf, jnp.float32)
        bi_s[...] = jnp.zeros(bi_s.shape, jnp.int32)

    zd = zd_s[...]
    cc = cb_ref[...]
    cn = jnp.sum(cc * cc, axis=1, keepdims=True)
    sc = lax.dot_general(cc, zd, (((1,), (0,)), ((), ())),
                         preferred_element_type=jnp.float32)
    dist = (sumz_s[...] - sc) + cn
    mn = jnp.min(dist, axis=0, keepdims=True)
    iota = lax.broadcasted_iota(jnp.int32, (chunk, 1), 0) + k * chunk
    cand = jnp.where(dist == mn, iota, jnp.int32(2 ** 30))
    am = jnp.min(cand, axis=0, keepdims=True)
    take = bd_s[...] <= mn
    bi_s[...] = jnp.where(take, bi_s[...], am)
    bd_s[...] = jnp.where(take, bd_s[...], mn
                          ).astype(jnp.bfloat16).astype(jnp.float32)

    @pl.when(k == nchunk - 1)
    def _():
        idx_ref[0] = bi_s[...]


def _gather_body(rows_per_w, idx_rows_per_w,
                 idx_hbm, table_hbm, out_hbm, idx_v, rows_v, sem):
    wid = lax.axis_index("s") * _SC_CORES + lax.axis_index("c")
    pltpu.sync_copy(idx_hbm.at[pl.ds(wid * idx_rows_per_w, idx_rows_per_w)],
                    idx_v)
    for j in range(idx_rows_per_w):
        pltpu.async_copy(table_hbm.at[idx_v.at[j]],
                         rows_v.at[pl.ds(j * _IDXW, _IDXW)], sem).wait()
    pltpu.sync_copy(rows_v, out_hbm.at[pl.ds(wid * rows_per_w, rows_per_w)])


def _decode_body(nb, ndtok,
                 zq_ref, ze_ref, vout_ref, gout_ref, bout_ref,
                 out_ref, loss_ref):
    b = pl.program_id(0)
    v = vout_ref[...]
    norm = jnp.sqrt(jnp.sum(v * v, axis=1, keepdims=True))
    w = gout_ref[...] * v / norm
    zq = zq_ref[0][:, :v.shape[1]]
    out = lax.dot_general(w, zq, (((1,), (1,)), ((), ())))
    out_ref[0] = out + bout_ref[...]

    ze = ze_ref[0]
    m = lax.dot_general(ze, zq, (((1,), (0,)), ((), ())))
    d = m.shape[0]
    eye = (lax.broadcasted_iota(jnp.int32, (d, d), 0)
           == lax.broadcasted_iota(jnp.int32, (d, d), 1))
    cross = jnp.sum(jnp.where(eye, m, 0.0))
    part = jnp.sum(ze * ze) + jnp.sum(zq * zq) - 2.0 * cross

    @pl.when(b == 0)
    def _():
        loss_ref[...] = jnp.zeros((1, 1), jnp.float32)

    loss_ref[...] = loss_ref[...] + jnp.reshape(part, (1, 1))

    @pl.when(b == nb - 1)
    def _():
        mean = loss_ref[...] / jnp.float32(ndtok)
        loss_ref[...] = mean + 0.25 * mean


def kernel(z, in_proj_v, in_proj_g, in_proj_b,
           out_proj_v, out_proj_g, out_proj_b, codebook):
    B, CIN, H, W = z.shape
    CB, D = codebook.shape
    NTOK = H * W
    CHUNK = 2048
    NCHUNK = CB // CHUNK
    TB = 512
    TSPLIT = NTOK // TB
    NT = B * TSPLIT

    z3 = z.reshape(B, CIN, NTOK)
    vin = in_proj_v.reshape(D, CIN)
    gin = in_proj_g.reshape(D, 1)
    bin_ = in_proj_b.reshape(D, 1)
    vout = out_proj_v.reshape(CIN, D)
    gout = out_proj_g.reshape(CIN, 1)
    bout = out_proj_b.reshape(CIN, 1)

    ze, idx = pl.pallas_call(
        functools.partial(_encode_body, NCHUNK, CHUNK, TB),
        grid=(NT, NCHUNK),
        in_specs=[
            pl.BlockSpec((1, CIN, TB), lambda t, k: (t // TSPLIT, 0, t % TSPLIT)),
            pl.BlockSpec((D, CIN), lambda t, k: (0, 0)),
            pl.BlockSpec((D, 1), lambda t, k: (0, 0)),
            pl.BlockSpec((D, 1), lambda t, k: (0, 0)),
            pl.BlockSpec((CHUNK, D), lambda t, k: (k, 0)),
        ],
        out_specs=[
            pl.BlockSpec((1, D, TB), lambda t, k: (t // TSPLIT, 0, t % TSPLIT)),
            pl.BlockSpec((1, 1, TB), lambda t, k: (t // TSPLIT, 0, t % TSPLIT)),
        ],
        out_shape=[
            jax.ShapeDtypeStruct((B, D, NTOK), jnp.float32),
            jax.ShapeDtypeStruct((B, 1, NTOK), jnp.int32),
        ],
        scratch_shapes=[
            pltpu.VMEM((D, TB), jnp.bfloat16),
            pltpu.VMEM((1, TB), jnp.float32),
            pltpu.VMEM((1, TB), jnp.float32),
            pltpu.VMEM((1, TB), jnp.int32),
        ],
        compiler_params=pltpu.CompilerParams(
            dimension_semantics=("parallel", "arbitrary")),
    )(z3, vin, gin, bin_, codebook)

    ntotal = B * NTOK
    rows_per_w = ntotal // _NW
    idx_rows_per_w = rows_per_w // _IDXW
    idx2 = idx.reshape(ntotal // _IDXW, _IDXW)
    cb_pad = jnp.pad(codebook, ((0, 0), (0, 128 - D)))

    zq = pl.kernel(
        functools.partial(_gather_body, rows_per_w, idx_rows_per_w),
        mesh=plsc.VectorSubcoreMesh(core_axis_name="c", subcore_axis_name="s"),
        out_type=jax.ShapeDtypeStruct((ntotal, 128), jnp.float32),
        scratch_types=[
            pltpu.VMEM((idx_rows_per_w, _IDXW), jnp.int32),
            pltpu.VMEM((rows_per_w, 128), jnp.float32),
            pltpu.SemaphoreType.DMA,
        ],
    )(idx2, cb_pad)

    zq3 = zq.reshape(B, NTOK, 128)
    out3, loss = pl.pallas_call(
        functools.partial(_decode_body, B, B * D * NTOK),
        grid=(B,),
        in_specs=[
            pl.BlockSpec((1, NTOK, 128), lambda b: (b, 0, 0)),
            pl.BlockSpec((1, D, NTOK), lambda b: (b, 0, 0)),
            pl.BlockSpec((CIN, D), lambda b: (0, 0)),
            pl.BlockSpec((CIN, 1), lambda b: (0, 0)),
            pl.BlockSpec((CIN, 1), lambda b: (0, 0)),
        ],
        out_specs=[
            pl.BlockSpec((1, CIN, NTOK), lambda b: (b, 0, 0)),
            pl.BlockSpec((1, 1), lambda b: (0, 0)),
        ],
        out_shape=[
            jax.ShapeDtypeStruct((B, CIN, NTOK), jnp.float32),
            jax.ShapeDtypeStruct((1, 1), jnp.float32),
        ],
        compiler_params=pltpu.CompilerParams(
            dimension_semantics=("arbitrary",)),
    )(zq3, ze, vout, gout, bout)

    out = out3.reshape(B, CIN, H, W)
    indices = idx.reshape(B, H, W)
    vq_loss = loss[0, 0]
    return out, indices, vq_loss


# speedup vs baseline: 1.1337x; 1.1337x over previous
import functools

import jax
import jax.numpy as jnp
from jax import lax
from jax.experimental import pallas as pl
from jax.experimental.pallas import tpu as pltpu
from jax.experimental.pallas import tpu_sc as plsc

_SC_CORES = 2
_SC_SUBCORES = 16
_NW = _SC_CORES * _SC_SUBCORES
_IDXW = 128


def _encode_body(nchunk, chunk, tb,
                 z_ref, vin_ref, gin_ref, bin_ref, cb_ref,
                 ze_ref, idx_ref,
                 zd_s, sumz_s, bd_s, bi_s):
    k = pl.program_id(1)

    @pl.when(k == 0)
    def _():
        v = vin_ref[...]
        norm = jnp.sqrt(jnp.sum(v * v, axis=1, keepdims=True))
        w = gin_ref[...] * v / norm
        ze = lax.dot_general(w, z_ref[0], (((1,), (0,)), ((), ())))
        ze = ze + bin_ref[...]
        ze_ref[0] = ze
        zd_s[...] = (2.0 * ze).astype(jnp.bfloat16)
        sumz_s[...] = jnp.sum(ze * ze, axis=0, keepdims=True)
        bd_s[...] = jnp.full(bd_s.shape, jnp.inf, jnp.float32)
        bi_s[...] = jnp.zeros(bi_s.shape, jnp.int32)

    zd = zd_s[...]
    cc = cb_ref[pl.ds(k * chunk, chunk), :]
    cn = jnp.sum(cc * cc, axis=1, keepdims=True)
    sc = lax.dot_general(cc, zd, (((1,), (0,)), ((), ())),
                         preferred_element_type=jnp.float32)
    dist = (sumz_s[...] - sc) + cn
    mn = jnp.min(dist, axis=0, keepdims=True)
    iota = (lax.broadcasted_iota(jnp.int32, (chunk, 1), 0).astype(jnp.float32)
            + (k * chunk).astype(jnp.float32))
    cand = jnp.where(dist == mn, iota, jnp.float32(3e38))
    am = jnp.min(cand, axis=0, keepdims=True).astype(jnp.int32)
    take = bd_s[...] <= mn
    bi_s[...] = jnp.where(take, bi_s[...], am)
    bd_s[...] = jnp.where(take, bd_s[...], mn
                          ).astype(jnp.bfloat16).astype(jnp.float32)

    @pl.when(k == nchunk - 1)
    def _():
        idx_ref[0] = bi_s[...]


def _gather_body(rows_per_w, idx_rows_per_w,
                 idx_hbm, table_hbm, out_hbm, idx_v, rows_v, sem):
    wid = lax.axis_index("s") * _SC_CORES + lax.axis_index("c")
    pltpu.sync_copy(idx_hbm.at[pl.ds(wid * idx_rows_per_w, idx_rows_per_w)],
                    idx_v)
    for j in range(idx_rows_per_w):
        pltpu.async_copy(table_hbm.at[idx_v.at[j]],
                         rows_v.at[pl.ds(j * _IDXW, _IDXW)], sem).wait()
    pltpu.sync_copy(rows_v, out_hbm.at[pl.ds(wid * rows_per_w, rows_per_w)])


def _decode_body(nb, ndtok,
                 zq_ref, ze_ref, vout_ref, gout_ref, bout_ref,
                 out_ref, loss_ref):
    b = pl.program_id(0)
    v = vout_ref[...]
    norm = jnp.sqrt(jnp.sum(v * v, axis=1, keepdims=True))
    w = gout_ref[...] * v / norm
    zq = zq_ref[0][:, :v.shape[1]]
    out = lax.dot_general(w, zq, (((1,), (1,)), ((), ())))
    out_ref[0] = out + bout_ref[...]

    ze = ze_ref[0]
    m = lax.dot_general(ze, zq, (((1,), (0,)), ((), ())))
    d = m.shape[0]
    eye = (lax.broadcasted_iota(jnp.int32, (d, d), 0)
           == lax.broadcasted_iota(jnp.int32, (d, d), 1))
    cross = jnp.sum(jnp.where(eye, m, 0.0))
    part = jnp.sum(ze * ze) + jnp.sum(zq * zq) - 2.0 * cross

    @pl.when(b == 0)
    def _():
        loss_ref[...] = jnp.zeros((1, 1), jnp.float32)

    loss_ref[...] = loss_ref[...] + jnp.reshape(part, (1, 1))

    @pl.when(b == nb - 1)
    def _():
        mean = loss_ref[...] / jnp.float32(ndtok)
        loss_ref[...] = mean + 0.25 * mean


def kernel(z, in_proj_v, in_proj_g, in_proj_b,
           out_proj_v, out_proj_g, out_proj_b, codebook):
    B, CIN, H, W = z.shape
    CB, D = codebook.shape
    NTOK = H * W
    CHUNK = 2048
    NCHUNK = CB // CHUNK
    TB = 1024
    TSPLIT = NTOK // TB
    NT = B * TSPLIT

    z3 = z.reshape(B, CIN, NTOK)
    vin = in_proj_v.reshape(D, CIN)
    gin = in_proj_g.reshape(D, 1)
    bin_ = in_proj_b.reshape(D, 1)
    vout = out_proj_v.reshape(CIN, D)
    gout = out_proj_g.reshape(CIN, 1)
    bout = out_proj_b.reshape(CIN, 1)

    ze, idx = pl.pallas_call(
        functools.partial(_encode_body, NCHUNK, CHUNK, TB),
        grid=(NT, NCHUNK),
        in_specs=[
            pl.BlockSpec((1, CIN, TB), lambda t, k: (t // TSPLIT, 0, t % TSPLIT)),
            pl.BlockSpec((D, CIN), lambda t, k: (0, 0)),
            pl.BlockSpec((D, 1), lambda t, k: (0, 0)),
            pl.BlockSpec((D, 1), lambda t, k: (0, 0)),
            pl.BlockSpec((CB, D), lambda t, k: (0, 0)),
        ],
        out_specs=[
            pl.BlockSpec((1, D, TB), lambda t, k: (t // TSPLIT, 0, t % TSPLIT)),
            pl.BlockSpec((1, 1, TB), lambda t, k: (t // TSPLIT, 0, t % TSPLIT)),
        ],
        out_shape=[
            jax.ShapeDtypeStruct((B, D, NTOK), jnp.float32),
            jax.ShapeDtypeStruct((B, 1, NTOK), jnp.int32),
        ],
        scratch_shapes=[
            pltpu.VMEM((D, TB), jnp.bfloat16),
            pltpu.VMEM((1, TB), jnp.float32),
            pltpu.VMEM((1, TB), jnp.float32),
            pltpu.VMEM((1, TB), jnp.int32),
        ],
        compiler_params=pltpu.CompilerParams(
            dimension_semantics=("parallel", "arbitrary")),
    )(z3, vin, gin, bin_, codebook)

    ntotal = B * NTOK
    rows_per_w = ntotal // _NW
    idx_rows_per_w = rows_per_w // _IDXW
    idx2 = idx.reshape(ntotal // _IDXW, _IDXW)
    cb_pad = jnp.pad(codebook, ((0, 0), (0, 128 - D)))

    zq = pl.kernel(
        functools.partial(_gather_body, rows_per_w, idx_rows_per_w),
        mesh=plsc.VectorSubcoreMesh(core_axis_name="c", subcore_axis_name="s"),
        out_type=jax.ShapeDtypeStruct((ntotal, 128), jnp.float32),
        scratch_types=[
            pltpu.VMEM((idx_rows_per_w, _IDXW), jnp.int32),
            pltpu.VMEM((rows_per_w, 128), jnp.float32),
            pltpu.SemaphoreType.DMA,
        ],
    )(idx2, cb_pad)

    zq3 = zq.reshape(B, NTOK, 128)
    out3, loss = pl.pallas_call(
        functools.partial(_decode_body, B, B * D * NTOK),
        grid=(B,),
        in_specs=[
            pl.BlockSpec((1, NTOK, 128), lambda b: (b, 0, 0)),
            pl.BlockSpec((1, D, NTOK), lambda b: (b, 0, 0)),
            pl.BlockSpec((CIN, D), lambda b: (0, 0)),
            pl.BlockSpec((CIN, 1), lambda b: (0, 0)),
            pl.BlockSpec((CIN, 1), lambda b: (0, 0)),
        ],
        out_specs=[
            pl.BlockSpec((1, CIN, NTOK), lambda b: (b, 0, 0)),
            pl.BlockSpec((1, 1), lambda b: (0, 0)),
        ],
        out_shape=[
            jax.ShapeDtypeStruct((B, CIN, NTOK), jnp.float32),
            jax.ShapeDtypeStruct((1, 1), jnp.float32),
        ],
        compiler_params=pltpu.CompilerParams(
            dimension_semantics=("arbitrary",)),
    )(zq3, ze, vout, gout, bout)

    out = out3.reshape(B, CIN, H, W)
    indices = idx.reshape(B, H, W)
    vq_loss = loss[0, 0]
    return out, indices, vq_loss


# speedup vs baseline: 1.1490x; 1.0136x over previous
import functools

import jax
import jax.numpy as jnp
from jax import lax
from jax.experimental import pallas as pl
from jax.experimental.pallas import tpu as pltpu
from jax.experimental.pallas import tpu_sc as plsc

_SC_CORES = 2
_SC_SUBCORES = 16
_NW = _SC_CORES * _SC_SUBCORES
_IDXW = 128


def _encode_body(nchunk, chunk, tb,
                 z_ref, vin_ref, gin_ref, bin_ref, cb_ref,
                 ze_ref, idx_ref,
                 zd_s, sumz_s, bd_s, bi_s):
    k = pl.program_id(1)

    @pl.when(k == 0)
    def _():
        v = vin_ref[...]
        norm = jnp.sqrt(jnp.sum(v * v, axis=1, keepdims=True))
        w = gin_ref[...] * v / norm
        ze = lax.dot_general(w, z_ref[0], (((1,), (0,)), ((), ())))
        ze = ze + bin_ref[...]
        ze_ref[0] = ze
        zd_s[...] = (2.0 * ze).astype(jnp.bfloat16)
        sumz_s[...] = jnp.sum(ze * ze, axis=0, keepdims=True)
        bd_s[...] = jnp.full(bd_s.shape, jnp.inf, jnp.float32)
        bi_s[...] = jnp.zeros(bi_s.shape, jnp.int32)

    zd = zd_s[...]
    sumz = sumz_s[...]
    sub = 256
    iota0 = lax.broadcasted_iota(jnp.int32, (sub, 1), 0).astype(jnp.float32)
    wv = None
    for s in range(chunk // sub):
        cc = cb_ref[pl.ds(k * chunk + s * sub, sub), :]
        cn = jnp.sum(cc * cc, axis=1, keepdims=True)
        sc = lax.dot_general(cc, zd, (((1,), (0,)), ((), ())),
                             preferred_element_type=jnp.float32)
        d = (sumz - sc) + cn
        mn = jnp.min(d, axis=0, keepdims=True)
        iota = iota0 + (k * chunk + s * sub).astype(jnp.float32)
        am = jnp.min(jnp.where(d == mn, iota, jnp.float32(3e38)),
                     axis=0, keepdims=True)
        if wv is None:
            wv, wi = mn, am
        else:
            tk = (wv < mn) | ((wv == mn) & (wi <= am))
            wv = jnp.where(tk, wv, mn)
            wi = jnp.where(tk, wi, am)
    am = wi.astype(jnp.int32)
    take = bd_s[...] <= wv
    bi_s[...] = jnp.where(take, bi_s[...], am)
    bd_s[...] = jnp.where(take, bd_s[...], wv
                          ).astype(jnp.bfloat16).astype(jnp.float32)

    @pl.when(k == nchunk - 1)
    def _():
        idx_ref[0] = bi_s[...]


def _gather_body(rows_per_w, idx_rows_per_w,
                 idx_hbm, table_hbm, out_hbm, idx_v, rows_v, sem):
    wid = lax.axis_index("s") * _SC_CORES + lax.axis_index("c")
    pltpu.sync_copy(idx_hbm.at[pl.ds(wid * idx_rows_per_w, idx_rows_per_w)],
                    idx_v)
    for j in range(idx_rows_per_w):
        pltpu.async_copy(table_hbm.at[idx_v.at[j]],
                         rows_v.at[pl.ds(j * _IDXW, _IDXW)], sem).wait()
    pltpu.sync_copy(rows_v, out_hbm.at[pl.ds(wid * rows_per_w, rows_per_w)])


def _decode_body(nb, ndtok,
                 zq_ref, ze_ref, vout_ref, gout_ref, bout_ref,
                 out_ref, loss_ref):
    b = pl.program_id(0)
    v = vout_ref[...]
    norm = jnp.sqrt(jnp.sum(v * v, axis=1, keepdims=True))
    w = gout_ref[...] * v / norm
    zq = zq_ref[0][:, :v.shape[1]]
    out = lax.dot_general(w, zq, (((1,), (1,)), ((), ())))
    out_ref[0] = out + bout_ref[...]

    ze = ze_ref[0]
    m = lax.dot_general(ze, zq, (((1,), (0,)), ((), ())))
    d = m.shape[0]
    eye = (lax.broadcasted_iota(jnp.int32, (d, d), 0)
           == lax.broadcasted_iota(jnp.int32, (d, d), 1))
    cross = jnp.sum(jnp.where(eye, m, 0.0))
    part = jnp.sum(ze * ze) + jnp.sum(zq * zq) - 2.0 * cross

    @pl.when(b == 0)
    def _():
        loss_ref[...] = jnp.zeros((1, 1), jnp.float32)

    loss_ref[...] = loss_ref[...] + jnp.reshape(part, (1, 1))

    @pl.when(b == nb - 1)
    def _():
        mean = loss_ref[...] / jnp.float32(ndtok)
        loss_ref[...] = mean + 0.25 * mean


def kernel(z, in_proj_v, in_proj_g, in_proj_b,
           out_proj_v, out_proj_g, out_proj_b, codebook):
    B, CIN, H, W = z.shape
    CB, D = codebook.shape
    NTOK = H * W
    CHUNK = 2048
    NCHUNK = CB // CHUNK
    TB = 1024
    TSPLIT = NTOK // TB
    NT = B * TSPLIT

    z3 = z.reshape(B, CIN, NTOK)
    vin = in_proj_v.reshape(D, CIN)
    gin = in_proj_g.reshape(D, 1)
    bin_ = in_proj_b.reshape(D, 1)
    vout = out_proj_v.reshape(CIN, D)
    gout = out_proj_g.reshape(CIN, 1)
    bout = out_proj_b.reshape(CIN, 1)

    ze, idx = pl.pallas_call(
        functools.partial(_encode_body, NCHUNK, CHUNK, TB),
        grid=(NT, NCHUNK),
        in_specs=[
            pl.BlockSpec((1, CIN, TB), lambda t, k: (t // TSPLIT, 0, t % TSPLIT)),
            pl.BlockSpec((D, CIN), lambda t, k: (0, 0)),
            pl.BlockSpec((D, 1), lambda t, k: (0, 0)),
            pl.BlockSpec((D, 1), lambda t, k: (0, 0)),
            pl.BlockSpec((CB, D), lambda t, k: (0, 0)),
        ],
        out_specs=[
            pl.BlockSpec((1, D, TB), lambda t, k: (t // TSPLIT, 0, t % TSPLIT)),
            pl.BlockSpec((1, 1, TB), lambda t, k: (t // TSPLIT, 0, t % TSPLIT)),
        ],
        out_shape=[
            jax.ShapeDtypeStruct((B, D, NTOK), jnp.float32),
            jax.ShapeDtypeStruct((B, 1, NTOK), jnp.int32),
        ],
        scratch_shapes=[
            pltpu.VMEM((D, TB), jnp.bfloat16),
            pltpu.VMEM((1, TB), jnp.float32),
            pltpu.VMEM((1, TB), jnp.float32),
            pltpu.VMEM((1, TB), jnp.int32),
        ],
        compiler_params=pltpu.CompilerParams(
            dimension_semantics=("parallel", "arbitrary")),
    )(z3, vin, gin, bin_, codebook)

    ntotal = B * NTOK
    rows_per_w = ntotal // _NW
    idx_rows_per_w = rows_per_w // _IDXW
    idx2 = idx.reshape(ntotal // _IDXW, _IDXW)
    cb_pad = jnp.pad(codebook, ((0, 0), (0, 128 - D)))

    zq = pl.kernel(
        functools.partial(_gather_body, rows_per_w, idx_rows_per_w),
        mesh=plsc.VectorSubcoreMesh(core_axis_name="c", subcore_axis_name="s"),
        out_type=jax.ShapeDtypeStruct((ntotal, 128), jnp.float32),
        scratch_types=[
            pltpu.VMEM((idx_rows_per_w, _IDXW), jnp.int32),
            pltpu.VMEM((rows_per_w, 128), jnp.float32),
            pltpu.SemaphoreType.DMA,
        ],
    )(idx2, cb_pad)

    zq3 = zq.reshape(B, NTOK, 128)
    out3, loss = pl.pallas_call(
        functools.partial(_decode_body, B, B * D * NTOK),
        grid=(B,),
        in_specs=[
            pl.BlockSpec((1, NTOK, 128), lambda b: (b, 0, 0)),
            pl.BlockSpec((1, D, NTOK), lambda b: (b, 0, 0)),
            pl.BlockSpec((CIN, D), lambda b: (0, 0)),
            pl.BlockSpec((CIN, 1), lambda b: (0, 0)),
            pl.BlockSpec((CIN, 1), lambda b: (0, 0)),
        ],
        out_specs=[
            pl.BlockSpec((1, CIN, NTOK), lambda b: (b, 0, 0)),
            pl.BlockSpec((1, 1), lambda b: (0, 0)),
        ],
        out_shape=[
            jax.ShapeDtypeStruct((B, CIN, NTOK), jnp.float32),
            jax.ShapeDtypeStruct((1, 1), jnp.float32),
        ],
        compiler_params=pltpu.CompilerParams(
            dimension_semantics=("arbitrary",)),
    )(zq3, ze, vout, gout, bout)

    out = out3.reshape(B, CIN, H, W)
    indices = idx.reshape(B, H, W)
    vq_loss = loss[0, 0]
    return out, indices, vq_loss


# speedup vs baseline: 1.2158x; 1.0581x over previous
import functools

import jax
import jax.numpy as jnp
from jax import lax
from jax.experimental import pallas as pl
from jax.experimental.pallas import tpu as pltpu
from jax.experimental.pallas import tpu_sc as plsc

_SC_CORES = 2
_SC_SUBCORES = 16
_NW = _SC_CORES * _SC_SUBCORES
_IDXW = 128


def _encode_body(nchunk, chunk, tb,
                 z_ref, vin_ref, gin_ref, bin_ref, cb_ref,
                 ze_ref, idx_ref):
    v = vin_ref[...]
    norm = jnp.sqrt(jnp.sum(v * v, axis=1, keepdims=True))
    w = gin_ref[...] * v / norm
    ze = lax.dot_general(w, z_ref[0], (((1,), (0,)), ((), ())))
    ze = ze + bin_ref[...]
    ze_ref[0] = ze
    zd = (2.0 * ze).astype(jnp.bfloat16)
    sumz = jnp.sum(ze * ze, axis=0, keepdims=True)

    sub = 256
    iota0 = lax.broadcasted_iota(jnp.int32, (sub, 1), 0).astype(jnp.float32)
    bd = bi = None
    for k in range(nchunk):
        wv = None
        for s in range(chunk // sub):
            cc = cb_ref[pl.ds(k * chunk + s * sub, sub), :]
            cn = jnp.sum(cc * cc, axis=1, keepdims=True)
            sc = lax.dot_general(cc, zd, (((1,), (0,)), ((), ())),
                                 preferred_element_type=jnp.float32)
            d = (sumz - sc) + cn
            mn = jnp.min(d, axis=0, keepdims=True)
            iota = iota0 + jnp.float32(k * chunk + s * sub)
            am = jnp.min(jnp.where(d == mn, iota, jnp.float32(3e38)),
                         axis=0, keepdims=True)
            if wv is None:
                wv, wi = mn, am
            else:
                tk = (wv < mn) | ((wv == mn) & (wi <= am))
                wv = jnp.where(tk, wv, mn)
                wi = jnp.where(tk, wi, am)
        if bd is None:
            bd = wv.astype(jnp.bfloat16).astype(jnp.float32)
            bi = wi
        else:
            take = bd <= wv
            bi = jnp.where(take, bi, wi)
            bd = jnp.where(take, bd, wv).astype(jnp.bfloat16).astype(jnp.float32)
    idx_ref[0] = bi.astype(jnp.int32)


def _gather_body(rows_per_w, idx_rows_per_w,
                 idx_hbm, table_hbm, out_hbm, idx_v, rows_v, sem):
    wid = lax.axis_index("s") * _SC_CORES + lax.axis_index("c")
    pltpu.sync_copy(idx_hbm.at[pl.ds(wid * idx_rows_per_w, idx_rows_per_w)],
                    idx_v)
    for j in range(idx_rows_per_w):
        pltpu.async_copy(table_hbm.at[idx_v.at[j]],
                         rows_v.at[pl.ds(j * _IDXW, _IDXW)], sem).wait()
    pltpu.sync_copy(rows_v, out_hbm.at[pl.ds(wid * rows_per_w, rows_per_w)])


def _decode_body(nb, ndtok,
                 zq_ref, ze_ref, vout_ref, gout_ref, bout_ref,
                 out_ref, loss_ref):
    b = pl.program_id(0)
    v = vout_ref[...]
    norm = jnp.sqrt(jnp.sum(v * v, axis=1, keepdims=True))
    w = gout_ref[...] * v / norm
    zq = zq_ref[0][:, :v.shape[1]]
    out = lax.dot_general(w, zq, (((1,), (1,)), ((), ())))
    out_ref[0] = out + bout_ref[...]

    ze = ze_ref[0]
    m = lax.dot_general(ze, zq, (((1,), (0,)), ((), ())))
    d = m.shape[0]
    eye = (lax.broadcasted_iota(jnp.int32, (d, d), 0)
           == lax.broadcasted_iota(jnp.int32, (d, d), 1))
    cross = jnp.sum(jnp.where(eye, m, 0.0))
    part = jnp.sum(ze * ze) + jnp.sum(zq * zq) - 2.0 * cross

    @pl.when(b == 0)
    def _():
        loss_ref[...] = jnp.zeros((1, 1), jnp.float32)

    loss_ref[...] = loss_ref[...] + jnp.reshape(part, (1, 1))

    @pl.when(b == nb - 1)
    def _():
        mean = loss_ref[...] / jnp.float32(ndtok)
        loss_ref[...] = mean + 0.25 * mean


def kernel(z, in_proj_v, in_proj_g, in_proj_b,
           out_proj_v, out_proj_g, out_proj_b, codebook):
    B, CIN, H, W = z.shape
    CB, D = codebook.shape
    NTOK = H * W
    CHUNK = 2048
    NCHUNK = CB // CHUNK
    TB = 1024
    TSPLIT = NTOK // TB
    NT = B * TSPLIT

    z3 = z.reshape(B, CIN, NTOK)
    vin = in_proj_v.reshape(D, CIN)
    gin = in_proj_g.reshape(D, 1)
    bin_ = in_proj_b.reshape(D, 1)
    vout = out_proj_v.reshape(CIN, D)
    gout = out_proj_g.reshape(CIN, 1)
    bout = out_proj_b.reshape(CIN, 1)

    ze, idx = pl.pallas_call(
        functools.partial(_encode_body, NCHUNK, CHUNK, TB),
        grid=(NT,),
        in_specs=[
            pl.BlockSpec((1, CIN, TB), lambda t: (t // TSPLIT, 0, t % TSPLIT)),
            pl.BlockSpec((D, CIN), lambda t: (0, 0)),
            pl.BlockSpec((D, 1), lambda t: (0, 0)),
            pl.BlockSpec((D, 1), lambda t: (0, 0)),
            pl.BlockSpec((CB, D), lambda t: (0, 0)),
        ],
        out_specs=[
            pl.BlockSpec((1, D, TB), lambda t: (t // TSPLIT, 0, t % TSPLIT)),
            pl.BlockSpec((1, 1, TB), lambda t: (t // TSPLIT, 0, t % TSPLIT)),
        ],
        out_shape=[
            jax.ShapeDtypeStruct((B, D, NTOK), jnp.float32),
            jax.ShapeDtypeStruct((B, 1, NTOK), jnp.int32),
        ],
        compiler_params=pltpu.CompilerParams(
            dimension_semantics=("parallel",)),
    )(z3, vin, gin, bin_, codebook)

    ntotal = B * NTOK
    rows_per_w = ntotal // _NW
    idx_rows_per_w = rows_per_w // _IDXW
    idx2 = idx.reshape(ntotal // _IDXW, _IDXW)
    cb_pad = jnp.pad(codebook, ((0, 0), (0, 128 - D)))

    zq = pl.kernel(
        functools.partial(_gather_body, rows_per_w, idx_rows_per_w),
        mesh=plsc.VectorSubcoreMesh(core_axis_name="c", subcore_axis_name="s"),
        out_type=jax.ShapeDtypeStruct((ntotal, 128), jnp.float32),
        scratch_types=[
            pltpu.VMEM((idx_rows_per_w, _IDXW), jnp.int32),
            pltpu.VMEM((rows_per_w, 128), jnp.float32),
            pltpu.SemaphoreType.DMA,
        ],
    )(idx2, cb_pad)

    zq3 = zq.reshape(B, NTOK, 128)
    out3, loss = pl.pallas_call(
        functools.partial(_decode_body, B, B * D * NTOK),
        grid=(B,),
        in_specs=[
            pl.BlockSpec((1, NTOK, 128), lambda b: (b, 0, 0)),
            pl.BlockSpec((1, D, NTOK), lambda b: (b, 0, 0)),
            pl.BlockSpec((CIN, D), lambda b: (0, 0)),
            pl.BlockSpec((CIN, 1), lambda b: (0, 0)),
            pl.BlockSpec((CIN, 1), lambda b: (0, 0)),
        ],
        out_specs=[
            pl.BlockSpec((1, CIN, NTOK), lambda b: (b, 0, 0)),
            pl.BlockSpec((1, 1), lambda b: (0, 0)),
        ],
        out_shape=[
            jax.ShapeDtypeStruct((B, CIN, NTOK), jnp.float32),
            jax.ShapeDtypeStruct((1, 1), jnp.float32),
        ],
        compiler_params=pltpu.CompilerParams(
            dimension_semantics=("arbitrary",)),
    )(zq3, ze, vout, gout, bout)

    out = out3.reshape(B, CIN, H, W)
    indices = idx.reshape(B, H, W)
    vq_loss = loss[0, 0]
    return out, indices, vq_loss


# speedup vs baseline: 1.2243x; 1.0070x over previous
import functools

import jax
import jax.numpy as jnp
from jax import lax
from jax.experimental import pallas as pl
from jax.experimental.pallas import tpu as pltpu
from jax.experimental.pallas import tpu_sc as plsc

_SC_CORES = 2
_SC_SUBCORES = 16
_NW = _SC_CORES * _SC_SUBCORES
_IDXW = 128


def _encode_body(nchunk, chunk, tb,
                 z_ref, vin_ref, gin_ref, bin_ref, cb_ref,
                 ze_ref, idx_ref):
    v = vin_ref[...]
    norm = jnp.sqrt(jnp.sum(v * v, axis=1, keepdims=True))
    w = gin_ref[...] * v / norm
    ze = lax.dot_general(w, z_ref[0], (((1,), (0,)), ((), ())))
    ze = ze + bin_ref[...]
    ze_ref[0] = ze
    zd = (2.0 * ze).astype(jnp.bfloat16)
    sumz = jnp.sum(ze * ze, axis=0, keepdims=True)

    sub = 256
    iota0 = lax.broadcasted_iota(jnp.int32, (sub, 1), 0).astype(jnp.float32)
    bd = bi = None
    for k in range(nchunk):
        wv = None
        for s in range(chunk // sub):
            cc = cb_ref[pl.ds(k * chunk + s * sub, sub), :]
            cn = jnp.sum(cc * cc, axis=1, keepdims=True)
            sc = lax.dot_general(cc, zd, (((1,), (0,)), ((), ())),
                                 preferred_element_type=jnp.float32)
            d = (sumz - sc) + cn
            mn = jnp.min(d, axis=0, keepdims=True)
            iota = iota0 + jnp.float32(k * chunk + s * sub)
            am = jnp.min(jnp.where(d == mn, iota, jnp.float32(3e38)),
                         axis=0, keepdims=True)
            if wv is None:
                wv, wi = mn, am
            else:
                tk = (wv < mn) | ((wv == mn) & (wi <= am))
                wv = jnp.where(tk, wv, mn)
                wi = jnp.where(tk, wi, am)
        if bd is None:
            bd = wv.astype(jnp.bfloat16).astype(jnp.float32)
            bi = wi
        else:
            take = bd <= wv
            bi = jnp.where(take, bi, wi)
            bd = jnp.where(take, bd, wv).astype(jnp.bfloat16).astype(jnp.float32)
    idx_ref[0] = bi.astype(jnp.int32)


def _gather_body(d, rows_per_w, idx_rows_per_w,
                 idx_hbm, table_hbm, out_hbm, idx_v, rows_v, sem):
    wid = lax.axis_index("s") * _SC_CORES + lax.axis_index("c")
    pltpu.sync_copy(idx_hbm.at[pl.ds(wid * idx_rows_per_w, idx_rows_per_w)],
                    idx_v)
    copies = [pltpu.async_copy(table_hbm.at[idx_v.at[j]],
                               rows_v.at[pl.ds(j * _IDXW, _IDXW)], sem)
              for j in range(idx_rows_per_w)]
    for c in copies:
        c.wait()
    pltpu.sync_copy(rows_v, out_hbm.at[pl.ds(wid * rows_per_w, rows_per_w)])


def _decode_body(nb, ndtok,
                 zq_ref, ze_ref, vout_ref, gout_ref, bout_ref,
                 out_ref, loss_ref):
    b = pl.program_id(0)
    v = vout_ref[...]
    norm = jnp.sqrt(jnp.sum(v * v, axis=1, keepdims=True))
    w = gout_ref[...] * v / norm
    zq = zq_ref[0][:, :v.shape[1]]
    out = lax.dot_general(w, zq, (((1,), (1,)), ((), ())))
    out_ref[0] = out + bout_ref[...]

    ze = ze_ref[0]
    m = lax.dot_general(ze, zq, (((1,), (0,)), ((), ())))
    d = m.shape[0]
    eye = (lax.broadcasted_iota(jnp.int32, (d, d), 0)
           == lax.broadcasted_iota(jnp.int32, (d, d), 1))
    cross = jnp.sum(jnp.where(eye, m, 0.0))
    part = jnp.sum(ze * ze) + jnp.sum(zq * zq) - 2.0 * cross

    @pl.when(b == 0)
    def _():
        loss_ref[...] = jnp.zeros((1, 1), jnp.float32)

    loss_ref[...] = loss_ref[...] + jnp.reshape(part, (1, 1))

    @pl.when(b == nb - 1)
    def _():
        mean = loss_ref[...] / jnp.float32(ndtok)
        loss_ref[...] = mean + 0.25 * mean


def kernel(z, in_proj_v, in_proj_g, in_proj_b,
           out_proj_v, out_proj_g, out_proj_b, codebook):
    B, CIN, H, W = z.shape
    CB, D = codebook.shape
    NTOK = H * W
    CHUNK = 2048
    NCHUNK = CB // CHUNK
    TB = 1024
    TSPLIT = NTOK // TB
    NT = B * TSPLIT

    z3 = z.reshape(B, CIN, NTOK)
    vin = in_proj_v.reshape(D, CIN)
    gin = in_proj_g.reshape(D, 1)
    bin_ = in_proj_b.reshape(D, 1)
    vout = out_proj_v.reshape(CIN, D)
    gout = out_proj_g.reshape(CIN, 1)
    bout = out_proj_b.reshape(CIN, 1)

    ze, idx = pl.pallas_call(
        functools.partial(_encode_body, NCHUNK, CHUNK, TB),
        grid=(NT,),
        in_specs=[
            pl.BlockSpec((1, CIN, TB), lambda t: (t // TSPLIT, 0, t % TSPLIT)),
            pl.BlockSpec((D, CIN), lambda t: (0, 0)),
            pl.BlockSpec((D, 1), lambda t: (0, 0)),
            pl.BlockSpec((D, 1), lambda t: (0, 0)),
            pl.BlockSpec((CB, D), lambda t: (0, 0)),
        ],
        out_specs=[
            pl.BlockSpec((1, D, TB), lambda t: (t // TSPLIT, 0, t % TSPLIT)),
            pl.BlockSpec((1, 1, TB), lambda t: (t // TSPLIT, 0, t % TSPLIT)),
        ],
        out_shape=[
            jax.ShapeDtypeStruct((B, D, NTOK), jnp.float32),
            jax.ShapeDtypeStruct((B, 1, NTOK), jnp.int32),
        ],
        compiler_params=pltpu.CompilerParams(
            dimension_semantics=("parallel",)),
    )(z3, vin, gin, bin_, codebook)

    ntotal = B * NTOK
    rows_per_w = ntotal // _NW
    idx_rows_per_w = rows_per_w // _IDXW
    idx2 = idx.reshape(ntotal // _IDXW, _IDXW)
    cb_pad = jnp.pad(codebook, ((0, 0), (0, 128 - D)))

    zq = pl.kernel(
        functools.partial(_gather_body, D, rows_per_w, idx_rows_per_w),
        mesh=plsc.VectorSubcoreMesh(core_axis_name="c", subcore_axis_name="s"),
        out_type=jax.ShapeDtypeStruct((ntotal, 128), jnp.float32),
        scratch_types=[
            pltpu.VMEM((idx_rows_per_w, _IDXW), jnp.int32),
            pltpu.VMEM((rows_per_w, 128), jnp.float32),
            pltpu.SemaphoreType.DMA,
        ],
    )(idx2, cb_pad)

    zq3 = zq.reshape(B, NTOK, 128)
    out3, loss = pl.pallas_call(
        functools.partial(_decode_body, B, B * D * NTOK),
        grid=(B,),
        in_specs=[
            pl.BlockSpec((1, NTOK, 128), lambda b: (b, 0, 0)),
            pl.BlockSpec((1, D, NTOK), lambda b: (b, 0, 0)),
            pl.BlockSpec((CIN, D), lambda b: (0, 0)),
            pl.BlockSpec((CIN, 1), lambda b: (0, 0)),
            pl.BlockSpec((CIN, 1), lambda b: (0, 0)),
        ],
        out_specs=[
            pl.BlockSpec((1, CIN, NTOK), lambda b: (b, 0, 0)),
            pl.BlockSpec((1, 1), lambda b: (0, 0)),
        ],
        out_shape=[
            jax.ShapeDtypeStruct((B, CIN, NTOK), jnp.float32),
            jax.ShapeDtypeStruct((1, 1), jnp.float32),
        ],
        compiler_params=pltpu.CompilerParams(
            dimension_semantics=("arbitrary",)),
    )(zq3, ze, vout, gout, bout)

    out = out3.reshape(B, CIN, H, W)
    indices = idx.reshape(B, H, W)
    vq_loss = loss[0, 0]
    return out, indices, vq_loss


# speedup vs baseline: 1.2349x; 1.0087x over previous
import functools

import jax
import jax.numpy as jnp
from jax import lax
from jax.experimental import pallas as pl
from jax.experimental.pallas import tpu as pltpu
from jax.experimental.pallas import tpu_sc as plsc

_SC_CORES = 2
_SC_SUBCORES = 16
_NW = _SC_CORES * _SC_SUBCORES
_IDXW = 128


def _encode_body(nchunk, chunk, tb,
                 z_ref, vin_ref, gin_ref, bin_ref, cb_ref,
                 ze_ref, idx_ref, cbp_ref):
    @pl.when(pl.program_id(0) == 0)
    def _():
        cbp_ref[:, :cb_ref.shape[1]] = cb_ref[...]
        cbp_ref[:, cb_ref.shape[1]:] = jnp.zeros(
            (cb_ref.shape[0], 128 - cb_ref.shape[1]), jnp.float32)
    v = vin_ref[...]
    norm = jnp.sqrt(jnp.sum(v * v, axis=1, keepdims=True))
    w = gin_ref[...] * v / norm
    ze = lax.dot_general(w, z_ref[0], (((1,), (0,)), ((), ())))
    ze = ze + bin_ref[...]
    ze_ref[0] = ze
    zd = (2.0 * ze).astype(jnp.bfloat16)
    sumz = jnp.sum(ze * ze, axis=0, keepdims=True)

    sub = 512
    iota0 = lax.broadcasted_iota(jnp.int32, (sub, 1), 0).astype(jnp.float32)
    bd = bi = None
    for k in range(nchunk):
        wv = None
        for s in range(chunk // sub):
            cc = cb_ref[pl.ds(k * chunk + s * sub, sub), :]
            cn = jnp.sum(cc * cc, axis=1, keepdims=True)
            sc = lax.dot_general(cc, zd, (((1,), (0,)), ((), ())),
                                 preferred_element_type=jnp.float32)
            d = (sumz - sc) + cn
            mn = jnp.min(d, axis=0, keepdims=True)
            iota = iota0 + jnp.float32(k * chunk + s * sub)
            am = jnp.min(jnp.where(d == mn, iota, jnp.float32(3e38)),
                         axis=0, keepdims=True)
            if wv is None:
                wv, wi = mn, am
            else:
                tk = (wv < mn) | ((wv == mn) & (wi <= am))
                wv = jnp.where(tk, wv, mn)
                wi = jnp.where(tk, wi, am)
        if bd is None:
            bd = wv.astype(jnp.bfloat16).astype(jnp.float32)
            bi = wi
        else:
            take = bd <= wv
            bi = jnp.where(take, bi, wi)
            bd = jnp.where(take, bd, wv).astype(jnp.bfloat16).astype(jnp.float32)
    idx_ref[0] = bi.astype(jnp.int32)


def _gather_body(d, rows_per_w, idx_rows_per_w,
                 idx_hbm, table_hbm, out_hbm, idx_v, rows_v, sem):
    wid = lax.axis_index("s") * _SC_CORES + lax.axis_index("c")
    pltpu.sync_copy(idx_hbm.at[pl.ds(wid * idx_rows_per_w, idx_rows_per_w)],
                    idx_v)
    copies = [pltpu.async_copy(table_hbm.at[idx_v.at[j]],
                               rows_v.at[pl.ds(j * _IDXW, _IDXW)], sem)
              for j in range(idx_rows_per_w)]
    for c in copies:
        c.wait()
    pltpu.sync_copy(rows_v, out_hbm.at[pl.ds(wid * rows_per_w, rows_per_w)])


def _decode_body(nb, ndtok,
                 zq_ref, ze_ref, vout_ref, gout_ref, bout_ref,
                 out_ref, loss_ref):
    b = pl.program_id(0)
    v = vout_ref[...]
    norm = jnp.sqrt(jnp.sum(v * v, axis=1, keepdims=True))
    w = gout_ref[...] * v / norm
    zq = zq_ref[0][:, :v.shape[1]]
    out = lax.dot_general(w, zq, (((1,), (1,)), ((), ())))
    out_ref[0] = out + bout_ref[...]

    ze = ze_ref[0]
    m = lax.dot_general(ze, zq, (((1,), (0,)), ((), ())))
    d = m.shape[0]
    eye = (lax.broadcasted_iota(jnp.int32, (d, d), 0)
           == lax.broadcasted_iota(jnp.int32, (d, d), 1))
    cross = jnp.sum(jnp.where(eye, m, 0.0))
    part = jnp.sum(ze * ze) + jnp.sum(zq * zq) - 2.0 * cross

    @pl.when(b == 0)
    def _():
        loss_ref[...] = jnp.zeros((1, 1), jnp.float32)

    loss_ref[...] = loss_ref[...] + jnp.reshape(part, (1, 1))

    @pl.when(b == nb - 1)
    def _():
        mean = loss_ref[...] / jnp.float32(ndtok)
        loss_ref[...] = mean + 0.25 * mean


def kernel(z, in_proj_v, in_proj_g, in_proj_b,
           out_proj_v, out_proj_g, out_proj_b, codebook):
    B, CIN, H, W = z.shape
    CB, D = codebook.shape
    NTOK = H * W
    CHUNK = 2048
    NCHUNK = CB // CHUNK
    TB = 1024
    TSPLIT = NTOK // TB
    NT = B * TSPLIT

    z3 = z.reshape(B, CIN, NTOK)
    vin = in_proj_v.reshape(D, CIN)
    gin = in_proj_g.reshape(D, 1)
    bin_ = in_proj_b.reshape(D, 1)
    vout = out_proj_v.reshape(CIN, D)
    gout = out_proj_g.reshape(CIN, 1)
    bout = out_proj_b.reshape(CIN, 1)

    ze, idx, cb_pad = pl.pallas_call(
        functools.partial(_encode_body, NCHUNK, CHUNK, TB),
        grid=(NT,),
        in_specs=[
            pl.BlockSpec((1, CIN, TB), lambda t: (t // TSPLIT, 0, t % TSPLIT)),
            pl.BlockSpec((D, CIN), lambda t: (0, 0)),
            pl.BlockSpec((D, 1), lambda t: (0, 0)),
            pl.BlockSpec((D, 1), lambda t: (0, 0)),
            pl.BlockSpec((CB, D), lambda t: (0, 0)),
        ],
        out_specs=[
            pl.BlockSpec((1, D, TB), lambda t: (t // TSPLIT, 0, t % TSPLIT)),
            pl.BlockSpec((1, 1, TB), lambda t: (t // TSPLIT, 0, t % TSPLIT)),
            pl.BlockSpec((CB, 128), lambda t: (0, 0)),
        ],
        out_shape=[
            jax.ShapeDtypeStruct((B, D, NTOK), jnp.float32),
            jax.ShapeDtypeStruct((B, 1, NTOK), jnp.int32),
            jax.ShapeDtypeStruct((CB, 128), jnp.float32),
        ],
        compiler_params=pltpu.CompilerParams(
            dimension_semantics=("arbitrary",)),
    )(z3, vin, gin, bin_, codebook)

    ntotal = B * NTOK
    rows_per_w = ntotal // _NW
    idx_rows_per_w = rows_per_w // _IDXW
    idx2 = idx.reshape(ntotal // _IDXW, _IDXW)

    zq = pl.kernel(
        functools.partial(_gather_body, D, rows_per_w, idx_rows_per_w),
        mesh=plsc.VectorSubcoreMesh(core_axis_name="c", subcore_axis_name="s"),
        out_type=jax.ShapeDtypeStruct((ntotal, 128), jnp.float32),
        scratch_types=[
            pltpu.VMEM((idx_rows_per_w, _IDXW), jnp.int32),
            pltpu.VMEM((rows_per_w, 128), jnp.float32),
            pltpu.SemaphoreType.DMA,
        ],
    )(idx2, cb_pad)

    zq3 = zq.reshape(B, NTOK, 128)
    out3, loss = pl.pallas_call(
        functools.partial(_decode_body, B, B * D * NTOK),
        grid=(B,),
        in_specs=[
            pl.BlockSpec((1, NTOK, 128), lambda b: (b, 0, 0)),
            pl.BlockSpec((1, D, NTOK), lambda b: (b, 0, 0)),
            pl.BlockSpec((CIN, D), lambda b: (0, 0)),
            pl.BlockSpec((CIN, 1), lambda b: (0, 0)),
            pl.BlockSpec((CIN, 1), lambda b: (0, 0)),
        ],
        out_specs=[
            pl.BlockSpec((1, CIN, NTOK), lambda b: (b, 0, 0)),
            pl.BlockSpec((1, 1), lambda b: (0, 0)),
        ],
        out_shape=[
            jax.ShapeDtypeStruct((B, CIN, NTOK), jnp.float32),
            jax.ShapeDtypeStruct((1, 1), jnp.float32),
        ],
        compiler_params=pltpu.CompilerParams(
            dimension_semantics=("arbitrary",)),
    )(zq3, ze, vout, gout, bout)

    out = out3.reshape(B, CIN, H, W)
    indices = idx.reshape(B, H, W)
    vq_loss = loss[0, 0]
    return out, indices, vq_loss
